# Initial kernel scaffold; baseline (speedup 1.0000x reference)
import jax
import jax.numpy as jnp
from jax.experimental import pallas as pl


def kernel(x, adj_matrix, degree_norm, num_avg, W, b):
    raise NotImplementedError("write your pallas kernel here")


# speedup vs baseline: 1.9901x; 1.9901x over previous
import jax
import jax.numpy as jnp
from jax.experimental import pallas as pl


def _proj_kernel(x_ref, w_ref, b_ref, h_ref):
    h_ref[...] = (
        jnp.dot(x_ref[...], w_ref[...], preferred_element_type=jnp.float32)
        + b_ref[...]
    )


def _hop_kernel(adj_ref, agg_ref, d_ref, out_ref):
    a = jnp.dot(adj_ref[...], agg_ref[...], preferred_element_type=jnp.float32)
    out_ref[...] = a * d_ref[...]


def _last_hop_kernel(adj_ref, agg_ref, h_ref, d_ref, out_ref):
    a = jnp.dot(adj_ref[...], agg_ref[...], preferred_element_type=jnp.float32)
    out_ref[...] = jnp.maximum(h_ref[...] - a * d_ref[...], 0.0)


def _row_block(n: int) -> int:
    for bm in (400, 200, 1000, 80, 40, 16, 8):
        if n % bm == 0:
            return bm
    return n


def kernel(x, adj_matrix, degree_norm, num_avg, W, b):
    n, feat = x.shape
    hid = W.shape[1]
    b2 = b.reshape(1, hid)

    h = pl.pallas_call(
        _proj_kernel,
        out_shape=jax.ShapeDtypeStruct((n, hid), jnp.float32),
    )(x, W, b2)

    bm = _row_block(n)
    grid = (n // bm,)

    def hop(agg):
        return pl.pallas_call(
            _hop_kernel,
            grid=grid,
            in_specs=[
                pl.BlockSpec((bm, n), lambda i: (i, 0)),
                pl.BlockSpec((n, hid), lambda i: (0, 0)),
                pl.BlockSpec((bm, 1), lambda i: (i, 0)),
            ],
            out_specs=pl.BlockSpec((bm, hid), lambda i: (i, 0)),
            out_shape=jax.ShapeDtypeStruct((n, hid), jnp.float32),
        )(adj_matrix, agg, degree_norm)

    agg = jax.lax.fori_loop(0, num_avg - 1, lambda _, a: hop(a), h)

    out = pl.pallas_call(
        _last_hop_kernel,
        grid=grid,
        in_specs=[
            pl.BlockSpec((bm, n), lambda i: (i, 0)),
            pl.BlockSpec((n, hid), lambda i: (0, 0)),
            pl.BlockSpec((bm, hid), lambda i: (i, 0)),
            pl.BlockSpec((bm, 1), lambda i: (i, 0)),
        ],
        out_specs=pl.BlockSpec((bm, hid), lambda i: (i, 0)),
        out_shape=jax.ShapeDtypeStruct((n, hid), jnp.float32),
    )(adj_matrix, agg, h, degree_norm)
    return out
